# baseline (speedup 1.0000x reference)
import jax
import jax.numpy as jnp
from jax.experimental import pallas as pl
from jax.experimental.pallas import tpu as pltpu
from jax.experimental.pallas import tpu_sc as plsc

_NUM_CODES = 8192
_DIM = 256
_TOKENS = 8192
_TM = 1024
_NTILES = _TOKENS // _TM
_TB = 128
_CH = 128
_NS = 1
_GATHER_W = 128


def _dist_argmin_kernel(x_ref, embT_ref, idx_ref, loss_ref, esq_ref):
    i = pl.program_id(0)

    @pl.when(i == 0)
    def _init():
        e = embT_ref[...]
        esq_ref[...] = jnp.sum(e * e, axis=0, keepdims=True)
        loss_ref[...] = jnp.zeros_like(loss_ref)

    x = x_ref[...]
    xsq = jnp.sum(x * x, axis=1, keepdims=True)
    mm2 = jax.lax.dot_general(
        x + x, embT_ref[...],
        dimension_numbers=(((1,), (0,)), ((), ())),
        preferred_element_type=jnp.float32,
    )
    esq = esq_ref[...]

    idx_parts = []
    m_parts = []
    nch = _NUM_CODES // _CH
    big = jnp.float32(_NUM_CODES)
    for tb in range(_TM // _TB):
        r0 = tb * _TB
        xsq_b = xsq[r0:r0 + _TB, :]
        mrun = [None] * _NS
        irun = [None] * _NS
        for c in range(nch):
            s = c % _NS
            l0 = c * _CH
            dc = (xsq_b - mm2[r0:r0 + _TB, l0:l0 + _CH]) \
                + esq[:, l0:l0 + _CH]
            if mrun[s] is None:
                mrun[s] = dc
                irun[s] = jnp.full((_TB, _CH), float(c), jnp.float32)
            else:
                better = dc < mrun[s]
                mrun[s] = jnp.minimum(mrun[s], dc)
                irun[s] = jnp.where(better, jnp.float32(c), irun[s])
        mall = mrun[0]
        for s in range(1, _NS):
            mall = jnp.minimum(mall, mrun[s])
        iall = jnp.where(mrun[0] == mall, irun[0], big)
        for s in range(1, _NS):
            iall = jnp.minimum(
                iall, jnp.where(mrun[s] == mall, irun[s], big))
        m_b = jnp.min(mall, axis=1, keepdims=True)
        lane = jax.lax.broadcasted_iota(
            jnp.int32, (_TB, _CH), 1).astype(jnp.float32)
        jl = iall * jnp.float32(_CH) + lane
        ids = jnp.where(mall == m_b, jl, big)
        idx_parts.append(jnp.min(ids, axis=1))
        m_parts.append(m_b)
    idx_ref[...] = jnp.concatenate(idx_parts).astype(
        jnp.int32).reshape(1, 1, _TM)
    m = jnp.concatenate(m_parts, axis=0)
    loss_ref[...] += jnp.sum(m, keepdims=True).reshape(1, 1)

    @pl.when(i == pl.num_programs(0) - 1)
    def _finalize():
        loss_ref[...] = loss_ref[...] * (0.25 / float(_TOKENS * _DIM))


def _vq_tc(flat_x, embT):
    ntiles = flat_x.shape[0] // _TM
    return pl.pallas_call(
        _dist_argmin_kernel,
        grid=(ntiles,),
        in_specs=[
            pl.BlockSpec((_TM, _DIM), lambda i: (i, 0)),
            pl.BlockSpec((_DIM, _NUM_CODES), lambda i: (0, 0)),
        ],
        out_specs=[
            pl.BlockSpec((1, 1, _TM), lambda i: (i, 0, 0)),
            pl.BlockSpec((1, 1), lambda i: (0, 0)),
        ],
        out_shape=[
            jax.ShapeDtypeStruct((ntiles, 1, _TM), jnp.int32),
            jax.ShapeDtypeStruct((1, 1), jnp.float32),
        ],
        scratch_shapes=[pltpu.VMEM((1, _NUM_CODES), jnp.float32)],
    )(flat_x, embT)


def _sc_gather(emb, idx2d):
    n = idx2d.shape[1]

    @pl.kernel(
        out_type=jax.ShapeDtypeStruct((n, _DIM), emb.dtype),
        mesh=plsc.VectorSubcoreMesh(core_axis_name="core",
                                    subcore_axis_name="subcore"),
    )
    def _gather(x_hbm, i_hbm, o_hbm):
        def body(i_vmem, o_vmem):
            pltpu.sync_copy(x_hbm.at[i_vmem.at[0]], o_vmem)

        pltpu.emit_pipeline(
            body,
            grid=(n // _GATHER_W,),
            in_specs=[pl.BlockSpec((1, _GATHER_W), index_map=lambda i: (0, i))],
            out_specs=[pl.BlockSpec((_GATHER_W, _DIM),
                                    index_map=lambda i: (i, 0))],
            core_axis_name=("core", "subcore"),
            dimension_semantics=(pltpu.PARALLEL,),
        )(i_hbm, o_hbm)

    return _gather(emb, idx2d)


def kernel(x, embedding):
    B, C, H, W = x.shape
    flat_x = jnp.transpose(x, (0, 2, 3, 1)).reshape(-1, C)
    embT = embedding.T
    idx3, loss11 = _vq_tc(flat_x, embT)
    idx = idx3.reshape(-1)
    q_flat = _sc_gather(embedding, idx.reshape(1, -1))
    quantized = jnp.transpose(q_flat.reshape(B, H, W, C), (0, 3, 1, 2))
    return (loss11[0, 0], quantized, idx)


# speedup vs baseline: 1.0048x; 1.0048x over previous
import jax
import jax.numpy as jnp
from jax.experimental import pallas as pl
from jax.experimental.pallas import tpu as pltpu
from jax.experimental.pallas import tpu_sc as plsc

_NUM_CODES = 8192
_DIM = 256
_TOKENS = 8192
_TM = 512
_NTILES = _TOKENS // _TM
_TB = 128
_CH = 128
_NS = 1
_GATHER_W = 128


def _dist_argmin_kernel(x_ref, embT_ref, idx_ref, loss_ref, esq_ref):
    i = pl.program_id(0)

    @pl.when(i == 0)
    def _init():
        e = embT_ref[...]
        esq_ref[...] = jnp.sum(e * e, axis=0, keepdims=True)
        loss_ref[...] = jnp.zeros_like(loss_ref)

    x = x_ref[...]
    xsq = jnp.sum(x * x, axis=1, keepdims=True)
    mm2 = jax.lax.dot_general(
        x + x, embT_ref[...],
        dimension_numbers=(((1,), (0,)), ((), ())),
        preferred_element_type=jnp.float32,
    )
    esq = esq_ref[...]

    idx_parts = []
    m_parts = []
    nch = _NUM_CODES // _CH
    big = jnp.float32(_NUM_CODES)
    for tb in range(_TM // _TB):
        r0 = tb * _TB
        xsq_b = xsq[r0:r0 + _TB, :]
        mrun = [None] * _NS
        irun = [None] * _NS
        for c in range(nch):
            s = c % _NS
            l0 = c * _CH
            dc = (xsq_b - mm2[r0:r0 + _TB, l0:l0 + _CH]) \
                + esq[:, l0:l0 + _CH]
            if mrun[s] is None:
                mrun[s] = dc
                irun[s] = jnp.full((_TB, _CH), float(c), jnp.float32)
            else:
                better = dc < mrun[s]
                mrun[s] = jnp.minimum(mrun[s], dc)
                irun[s] = jnp.where(better, jnp.float32(c), irun[s])
        mall = mrun[0]
        for s in range(1, _NS):
            mall = jnp.minimum(mall, mrun[s])
        iall = jnp.where(mrun[0] == mall, irun[0], big)
        for s in range(1, _NS):
            iall = jnp.minimum(
                iall, jnp.where(mrun[s] == mall, irun[s], big))
        m_b = jnp.min(mall, axis=1, keepdims=True)
        lane = jax.lax.broadcasted_iota(
            jnp.int32, (_TB, _CH), 1).astype(jnp.float32)
        jl = iall * jnp.float32(_CH) + lane
        ids = jnp.where(mall == m_b, jl, big)
        idx_parts.append(jnp.min(ids, axis=1))
        m_parts.append(m_b)
    idx_ref[...] = jnp.concatenate(idx_parts).astype(
        jnp.int32).reshape(1, 1, _TM)
    m = jnp.concatenate(m_parts, axis=0)
    loss_ref[...] += jnp.sum(m, keepdims=True).reshape(1, 1)

    @pl.when(i == pl.num_programs(0) - 1)
    def _finalize():
        loss_ref[...] = loss_ref[...] * (0.25 / float(_TOKENS * _DIM))


def _vq_tc(flat_x, embT):
    ntiles = flat_x.shape[0] // _TM
    return pl.pallas_call(
        _dist_argmin_kernel,
        grid=(ntiles,),
        in_specs=[
            pl.BlockSpec((_TM, _DIM), lambda i: (i, 0)),
            pl.BlockSpec((_DIM, _NUM_CODES), lambda i: (0, 0)),
        ],
        out_specs=[
            pl.BlockSpec((1, 1, _TM), lambda i: (i, 0, 0)),
            pl.BlockSpec((1, 1), lambda i: (0, 0)),
        ],
        out_shape=[
            jax.ShapeDtypeStruct((ntiles, 1, _TM), jnp.int32),
            jax.ShapeDtypeStruct((1, 1), jnp.float32),
        ],
        scratch_shapes=[pltpu.VMEM((1, _NUM_CODES), jnp.float32)],
    )(flat_x, embT)


def _sc_gather(emb, idx2d):
    n = idx2d.shape[1]

    @pl.kernel(
        out_type=jax.ShapeDtypeStruct((n, _DIM), emb.dtype),
        mesh=plsc.VectorSubcoreMesh(core_axis_name="core",
                                    subcore_axis_name="subcore"),
    )
    def _gather(x_hbm, i_hbm, o_hbm):
        def body(i_vmem, o_vmem):
            pltpu.sync_copy(x_hbm.at[i_vmem.at[0]], o_vmem)

        pltpu.emit_pipeline(
            body,
            grid=(n // _GATHER_W,),
            in_specs=[pl.BlockSpec((1, _GATHER_W), index_map=lambda i: (0, i))],
            out_specs=[pl.BlockSpec((_GATHER_W, _DIM),
                                    index_map=lambda i: (i, 0))],
            core_axis_name=("core", "subcore"),
            dimension_semantics=(pltpu.PARALLEL,),
        )(i_hbm, o_hbm)

    return _gather(emb, idx2d)


def kernel(x, embedding):
    B, C, H, W = x.shape
    flat_x = jnp.transpose(x, (0, 2, 3, 1)).reshape(-1, C)
    embT = embedding.T
    idx3, loss11 = _vq_tc(flat_x, embT)
    idx = idx3.reshape(-1)
    q_flat = _sc_gather(embedding, idx.reshape(1, -1))
    quantized = jnp.transpose(q_flat.reshape(B, H, W, C), (0, 3, 1, 2))
    return (loss11[0, 0], quantized, idx)
